# baseline (speedup 1.0000x reference)
import functools

import jax
import jax.numpy as jnp
from jax import lax
from jax.experimental import pallas as pl
from jax.experimental.pallas import tpu as pltpu
from jax.experimental.pallas import tpu_sc as plsc

N = 4096
D = 256
K = 2048
H = N // 2

_BI = 512
_BP = 512
_BM = 512


def _rank_body(h_ref, w_ref, b_ref, r_ref, s2_ref, sc_ref, sr_ref):
    i = pl.program_id(0)

    @pl.when(i == 0)
    def _():
        b0 = b_ref[0, 0]
        sc_ref[...] = jax.nn.sigmoid(
            jnp.dot(h_ref[...], w_ref[...],
                    preferred_element_type=jnp.float32) + b0)
        sr_ref[...] = jax.nn.sigmoid(
            lax.dot_general(w_ref[...], h_ref[...], (((0,), (1,)), ((), ())),
                            preferred_element_type=jnp.float32) + b0)
        s2_ref[...] = sr_ref[...]

    s_i = sc_ref[pl.ds(i * _BI, _BI), :]
    s_j = sr_ref[...]
    jj = lax.broadcasted_iota(jnp.int32, (_BI, N), 1)
    ii = i * _BI + lax.broadcasted_iota(jnp.int32, (_BI, N), 0)
    ahead = (s_j > s_i) | ((s_j == s_i) & (jj < ii))
    r_ref[...] = jnp.sum(ahead.astype(jnp.int32), axis=1, keepdims=True)


def _inv_body(r_row_ref, s_row_ref, idx_ref, val_ref):
    p = pl.program_id(0)
    rr = r_row_ref[...]
    ss = s_row_ref[...]
    pp = p * _BI + lax.broadcasted_iota(jnp.int32, (_BI, N), 0)
    jj = lax.broadcasted_iota(jnp.int32, (_BI, N), 1)
    m = rr == pp
    idx_ref[...] = jnp.sum(jnp.where(m, jj, 0), axis=1, keepdims=True)
    val_ref[...] = jnp.sum(jnp.where(m, ss, 0.0), axis=1, keepdims=True)


_RB = 256


def _prep_body(g_ref, up_ref, utp_ref):
    i = pl.program_id(0)
    graw = g_ref[...]
    lo = (graw[:, :H] != 0).astype(jnp.int32)
    hi = (graw[:, H:] != 0).astype(jnp.int32)
    up_ref[...] = lo + 2 * hi
    u_bf = (graw != 0).astype(jnp.bfloat16)
    eye = (lax.broadcasted_iota(jnp.int32, (_RB, _RB), 0)
           == lax.broadcasted_iota(jnp.int32, (_RB, _RB), 1)
           ).astype(jnp.bfloat16)
    ut = lax.dot_general(u_bf, eye, (((0,), (0,)), ((), ())),
                         preferred_element_type=jnp.float32)
    uti = ut.astype(jnp.int32)
    cs = pl.ds((i % (H // _RB)) * _RB, _RB)

    @pl.when(i < H // _RB)
    def _():
        utp_ref[:, cs] = uti

    @pl.when(i >= H // _RB)
    def _():
        utp_ref[:, cs] += 2 * uti


def _mmf_body(r_ref, ct_ref, hg_ref, val_ref, g_ref, h_ref, deg_ref,
              rlo_ref, rhi_ref):
    j = pl.program_id(1)

    @pl.when(j == 0)
    def _():
        rp = r_ref[...]
        rlo_ref[...] = (rp & 1).astype(jnp.bfloat16)
        rhi_ref[...] = (rp >> 1).astype(jnp.bfloat16)

    cp = ct_ref[...]
    clo = (cp & 1).astype(jnp.bfloat16)
    chi = (cp >> 1).astype(jnp.bfloat16)
    dims = (((1,), (1,)), ((), ()))
    acc = lax.dot_general(rlo_ref[...], clo, dims,
                          preferred_element_type=jnp.float32)
    acc = acc + lax.dot_general(rhi_ref[...], chi, dims,
                                preferred_element_type=jnp.float32)
    bin_f = (acc != 0).astype(jnp.float32)
    part = jnp.sum(bin_f, axis=1, keepdims=True)

    @pl.when(j == 0)
    def _():
        deg_ref[...] = part
        g_ref[:, pl.ds(0, _BM)] = bin_f

    @pl.when((j != 0) & (j != 3))
    def _():
        deg_ref[...] += part
        g_ref[:, pl.ds(j * _BM, _BM)] = bin_f

    @pl.when(j == 3)
    def _():
        d = deg_ref[...] + part
        rec = 1.0 / jnp.where(d == 0, 1.0, d)
        g_ref[:, pl.ds(0, 3 * _BM)] *= rec
        g_ref[:, pl.ds(3 * _BM, _BM)] = bin_f * rec
        h_ref[...] = hg_ref[...] * val_ref[...]


def _make_sc_gather(nc, ns):
    nw = nc * ns
    rpw = K // nw
    chunk = rpw // 4
    mesh = plsc.VectorSubcoreMesh(core_axis_name="c", subcore_axis_name="s")

    @functools.partial(
        pl.kernel, mesh=mesh,
        out_type=[
            jax.ShapeDtypeStruct((K, H), jnp.int32),
            jax.ShapeDtypeStruct((K, H), jnp.int32),
            jax.ShapeDtypeStruct((K, D), jnp.float32),
        ],
        scratch_types=[
            pltpu.VMEM((rpw,), jnp.int32),
            pltpu.VMEM((chunk, H), jnp.int32),
            pltpu.VMEM((chunk, H), jnp.int32),
            pltpu.VMEM((rpw, D), jnp.float32),
            pltpu.SemaphoreType.DMA,
            pltpu.SemaphoreType.DMA,
            pltpu.SemaphoreType.DMA,
        ],
    )
    def sc_gather(up, utp, h, idx, rp_out, ctp_out, hg_out,
                  idx_v, buf0, buf1, hbuf, sem_g, sem_w0, sem_w1):
        wid = lax.axis_index("s") * nc + lax.axis_index("c")
        base = wid * rpw
        pltpu.sync_copy(idx.at[pl.ds(base, rpw)], idx_v)
        pltpu.async_copy(h.at[idx_v], hbuf, sem_g).wait()
        pltpu.sync_copy(hbuf, hg_out.at[pl.ds(base, rpw)])
        bufs = (buf0, buf1)
        wsems = (sem_w0, sem_w1)
        pending = [None, None]
        t = 0
        for src, dst in ((up, rp_out), (utp, ctp_out)):
            for c in range(4):
                sl = t % 2
                if pending[sl] is not None:
                    pending[sl].wait()
                iv = idx_v.at[pl.ds(c * chunk, chunk)]
                pltpu.async_copy(src.at[iv], bufs[sl], sem_g).wait()
                pending[sl] = pltpu.async_copy(
                    bufs[sl], dst.at[pl.ds(base + c * chunk, chunk)], wsems[sl])
                t += 1
        pending[0].wait()
        pending[1].wait()

    return sc_gather


def kernel(g, h, W, b):
    f32 = jnp.float32
    r, s2 = pl.pallas_call(
        _rank_body,
        grid=(N // _BI,),
        in_specs=[
            pl.BlockSpec((N, D), lambda i: (0, 0)),
            pl.BlockSpec((D, 1), lambda i: (0, 0)),
            pl.BlockSpec((1, 1), lambda i: (0, 0)),
        ],
        out_specs=[
            pl.BlockSpec((_BI, 1), lambda i: (i, 0)),
            pl.BlockSpec((1, N), lambda i: (0, 0)),
        ],
        out_shape=[
            jax.ShapeDtypeStruct((N, 1), jnp.int32),
            jax.ShapeDtypeStruct((1, N), f32),
        ],
        scratch_shapes=[
            pltpu.VMEM((N, 1), f32),
            pltpu.VMEM((1, N), f32),
        ],
    )(h, W, b.reshape(1, 1))
    r2 = r.reshape(1, N)

    idxf, vals = pl.pallas_call(
        _inv_body,
        grid=(K // _BI,),
        in_specs=[
            pl.BlockSpec((1, N), lambda i: (0, 0)),
            pl.BlockSpec((1, N), lambda i: (0, 0)),
        ],
        out_specs=[
            pl.BlockSpec((_BI, 1), lambda i: (i, 0)),
            pl.BlockSpec((_BI, 1), lambda i: (i, 0)),
        ],
        out_shape=[
            jax.ShapeDtypeStruct((K, 1), jnp.int32),
            jax.ShapeDtypeStruct((K, 1), f32),
        ],
    )(r2, s2)
    idx = idxf.reshape(K)

    up, utp = pl.pallas_call(
        _prep_body,
        grid=(N // _RB,),
        in_specs=[
            pl.BlockSpec((_RB, N), lambda i: (i, 0)),
        ],
        out_specs=[
            pl.BlockSpec((_RB, H), lambda i: (i, 0)),
            pl.BlockSpec((N, H), lambda i: (0, 0)),
        ],
        out_shape=[
            jax.ShapeDtypeStruct((N, H), jnp.int32),
            jax.ShapeDtypeStruct((N, H), jnp.int32),
        ],
    )(g)

    info = plsc.get_sparse_core_info()
    sc_gather = _make_sc_gather(info.num_cores, info.num_subcores)
    Rp, CTp, Hg = sc_gather(up, utp, h, idx)

    g_new, new_h = pl.pallas_call(
        _mmf_body,
        grid=(K // _BM, K // _BM),
        in_specs=[
            pl.BlockSpec((_BM, H), lambda i, j: (i, 0)),
            pl.BlockSpec((_BM, H), lambda i, j: (j, 0)),
            pl.BlockSpec((_BM, D), lambda i, j: (i, 0)),
            pl.BlockSpec((_BM, 1), lambda i, j: (i, 0)),
        ],
        out_specs=[
            pl.BlockSpec((_BM, K), lambda i, j: (i, 0)),
            pl.BlockSpec((_BM, D), lambda i, j: (i, 0)),
        ],
        out_shape=[
            jax.ShapeDtypeStruct((K, K), f32),
            jax.ShapeDtypeStruct((K, D), f32),
        ],
        scratch_shapes=[
            pltpu.VMEM((_BM, 1), f32),
            pltpu.VMEM((_BM, H), jnp.bfloat16),
            pltpu.VMEM((_BM, H), jnp.bfloat16),
        ],
    )(Rp, CTp, Hg, vals)

    return g_new, new_h, idx


# speedup vs baseline: 1.0051x; 1.0051x over previous
import functools

import jax
import jax.numpy as jnp
from jax import lax
from jax.experimental import pallas as pl
from jax.experimental.pallas import tpu as pltpu
from jax.experimental.pallas import tpu_sc as plsc

N = 4096
D = 256
K = 2048
H = N // 2

_BI = 512
_BP = 512
_BM = 512


def _rank_body(h_ref, w_ref, b_ref, r_ref, s2_ref, sc_ref, sr_ref):
    i = pl.program_id(0)

    @pl.when(i == 0)
    def _():
        b0 = b_ref[0, 0]
        sc_ref[...] = jax.nn.sigmoid(
            jnp.dot(h_ref[...], w_ref[...],
                    preferred_element_type=jnp.float32) + b0)
        sr_ref[...] = jax.nn.sigmoid(
            lax.dot_general(w_ref[...], h_ref[...], (((0,), (1,)), ((), ())),
                            preferred_element_type=jnp.float32) + b0)
        s2_ref[...] = sr_ref[...]

    s_i = sc_ref[pl.ds(i * _BI, _BI), :]
    s_j = sr_ref[...]
    jj = lax.broadcasted_iota(jnp.int32, (_BI, N), 1)
    ii = i * _BI + lax.broadcasted_iota(jnp.int32, (_BI, N), 0)
    ahead = (s_j > s_i) | ((s_j == s_i) & (jj < ii))
    r_ref[...] = jnp.sum(ahead.astype(jnp.int32), axis=1, keepdims=True)


def _inv_body(r_row_ref, s_row_ref, idx_ref, val_ref):
    p = pl.program_id(0)
    rr = r_row_ref[...]
    ss = s_row_ref[...]
    pp = p * _BI + lax.broadcasted_iota(jnp.int32, (_BI, N), 0)
    jj = lax.broadcasted_iota(jnp.int32, (_BI, N), 1)
    m = rr == pp
    idx_ref[...] = jnp.sum(jnp.where(m, jj, 0), axis=1, keepdims=True)
    val_ref[...] = jnp.sum(jnp.where(m, ss, 0.0), axis=1, keepdims=True)


_RB = 256


def _prep_body(g_ref, up_ref, utp_ref):
    i = pl.program_id(0)
    graw = g_ref[...]
    lo = (graw[:, :H] != 0).astype(jnp.int32)
    hi = (graw[:, H:] != 0).astype(jnp.int32)
    up_ref[...] = lo + 2 * hi
    u_bf = (graw != 0).astype(jnp.bfloat16)
    eye = (lax.broadcasted_iota(jnp.int32, (_RB, _RB), 0)
           == lax.broadcasted_iota(jnp.int32, (_RB, _RB), 1)
           ).astype(jnp.bfloat16)
    ut = lax.dot_general(u_bf, eye, (((0,), (0,)), ((), ())),
                         preferred_element_type=jnp.float32)
    uti = ut.astype(jnp.int32)
    cs = pl.ds((i % (H // _RB)) * _RB, _RB)

    @pl.when(i < H // _RB)
    def _():
        utp_ref[:, cs] = uti

    @pl.when(i >= H // _RB)
    def _():
        utp_ref[:, cs] += 2 * uti


def _mmf_body(r_ref, ct_ref, hg_ref, val_ref, g_ref, h_ref, deg_ref):
    j = pl.program_id(1)
    rp = r_ref[...]
    cp = ct_ref[...]
    rlo = (rp & 1).astype(jnp.bfloat16)
    rhi = (rp >> 1).astype(jnp.bfloat16)
    clo = (cp & 1).astype(jnp.bfloat16)
    chi = (cp >> 1).astype(jnp.bfloat16)
    dims = (((1,), (1,)), ((), ()))
    acc = lax.dot_general(rlo, clo, dims,
                          preferred_element_type=jnp.float32)
    acc = acc + lax.dot_general(rhi, chi, dims,
                                preferred_element_type=jnp.float32)
    bin_f = (acc != 0).astype(jnp.float32)
    part = jnp.sum(bin_f, axis=1, keepdims=True)

    @pl.when(j == 0)
    def _():
        deg_ref[...] = part
        g_ref[:, pl.ds(0, _BM)] = bin_f

    @pl.when((j != 0) & (j != 3))
    def _():
        deg_ref[...] += part
        g_ref[:, pl.ds(j * _BM, _BM)] = bin_f

    @pl.when(j == 3)
    def _():
        d = deg_ref[...] + part
        rec = 1.0 / jnp.where(d == 0, 1.0, d)
        g_ref[:, pl.ds(0, 3 * _BM)] *= rec
        g_ref[:, pl.ds(3 * _BM, _BM)] = bin_f * rec
        h_ref[...] = hg_ref[...] * val_ref[...]


def _make_sc_gather(nc, ns):
    nw = nc * ns
    rpw = K // nw
    chunk = rpw // 4
    mesh = plsc.VectorSubcoreMesh(core_axis_name="c", subcore_axis_name="s")

    @functools.partial(
        pl.kernel, mesh=mesh,
        out_type=[
            jax.ShapeDtypeStruct((K, H), jnp.int32),
            jax.ShapeDtypeStruct((K, H), jnp.int32),
            jax.ShapeDtypeStruct((K, D), jnp.float32),
        ],
        scratch_types=[
            pltpu.VMEM((rpw,), jnp.int32),
            pltpu.VMEM((chunk, H), jnp.int32),
            pltpu.VMEM((chunk, H), jnp.int32),
            pltpu.VMEM((rpw, D), jnp.float32),
            pltpu.SemaphoreType.DMA,
            pltpu.SemaphoreType.DMA,
            pltpu.SemaphoreType.DMA,
        ],
    )
    def sc_gather(up, utp, h, idx, rp_out, ctp_out, hg_out,
                  idx_v, buf0, buf1, hbuf, sem_g, sem_w0, sem_w1):
        wid = lax.axis_index("s") * nc + lax.axis_index("c")
        base = wid * rpw
        pltpu.sync_copy(idx.at[pl.ds(base, rpw)], idx_v)
        pltpu.async_copy(h.at[idx_v], hbuf, sem_g).wait()
        pltpu.sync_copy(hbuf, hg_out.at[pl.ds(base, rpw)])
        bufs = (buf0, buf1)
        wsems = (sem_w0, sem_w1)
        pending = [None, None]
        t = 0
        for src, dst in ((up, rp_out), (utp, ctp_out)):
            for c in range(4):
                sl = t % 2
                if pending[sl] is not None:
                    pending[sl].wait()
                iv = idx_v.at[pl.ds(c * chunk, chunk)]
                pltpu.async_copy(src.at[iv], bufs[sl], sem_g).wait()
                pending[sl] = pltpu.async_copy(
                    bufs[sl], dst.at[pl.ds(base + c * chunk, chunk)], wsems[sl])
                t += 1
        pending[0].wait()
        pending[1].wait()

    return sc_gather


def kernel(g, h, W, b):
    f32 = jnp.float32
    r, s2 = pl.pallas_call(
        _rank_body,
        grid=(N // _BI,),
        in_specs=[
            pl.BlockSpec((N, D), lambda i: (0, 0)),
            pl.BlockSpec((D, 1), lambda i: (0, 0)),
            pl.BlockSpec((1, 1), lambda i: (0, 0)),
        ],
        out_specs=[
            pl.BlockSpec((_BI, 1), lambda i: (i, 0)),
            pl.BlockSpec((1, N), lambda i: (0, 0)),
        ],
        out_shape=[
            jax.ShapeDtypeStruct((N, 1), jnp.int32),
            jax.ShapeDtypeStruct((1, N), f32),
        ],
        scratch_shapes=[
            pltpu.VMEM((N, 1), f32),
            pltpu.VMEM((1, N), f32),
        ],
    )(h, W, b.reshape(1, 1))
    r2 = r.reshape(1, N)

    idxf, vals = pl.pallas_call(
        _inv_body,
        grid=(K // _BI,),
        in_specs=[
            pl.BlockSpec((1, N), lambda i: (0, 0)),
            pl.BlockSpec((1, N), lambda i: (0, 0)),
        ],
        out_specs=[
            pl.BlockSpec((_BI, 1), lambda i: (i, 0)),
            pl.BlockSpec((_BI, 1), lambda i: (i, 0)),
        ],
        out_shape=[
            jax.ShapeDtypeStruct((K, 1), jnp.int32),
            jax.ShapeDtypeStruct((K, 1), f32),
        ],
    )(r2, s2)
    idx = idxf.reshape(K)

    up, utp = pl.pallas_call(
        _prep_body,
        grid=(N // _RB,),
        in_specs=[
            pl.BlockSpec((_RB, N), lambda i: (i, 0)),
        ],
        out_specs=[
            pl.BlockSpec((_RB, H), lambda i: (i, 0)),
            pl.BlockSpec((N, H), lambda i: (0, 0)),
        ],
        out_shape=[
            jax.ShapeDtypeStruct((N, H), jnp.int32),
            jax.ShapeDtypeStruct((N, H), jnp.int32),
        ],
    )(g)

    info = plsc.get_sparse_core_info()
    sc_gather = _make_sc_gather(info.num_cores, info.num_subcores)
    Rp, CTp, Hg = sc_gather(up, utp, h, idx)

    g_new, new_h = pl.pallas_call(
        _mmf_body,
        grid=(K // _BM, K // _BM),
        in_specs=[
            pl.BlockSpec((_BM, H), lambda i, j: (i, 0)),
            pl.BlockSpec((_BM, H), lambda i, j: (j, 0)),
            pl.BlockSpec((_BM, D), lambda i, j: (i, 0)),
            pl.BlockSpec((_BM, 1), lambda i, j: (i, 0)),
        ],
        out_specs=[
            pl.BlockSpec((_BM, K), lambda i, j: (i, 0)),
            pl.BlockSpec((_BM, D), lambda i, j: (i, 0)),
        ],
        out_shape=[
            jax.ShapeDtypeStruct((K, K), f32),
            jax.ShapeDtypeStruct((K, D), f32),
        ],
        scratch_shapes=[
            pltpu.VMEM((_BM, 1), f32),
        ],
    )(Rp, CTp, Hg, vals)

    return g_new, new_h, idx


# speedup vs baseline: 1.0100x; 1.0049x over previous
import functools

import jax
import jax.numpy as jnp
from jax import lax
from jax.experimental import pallas as pl
from jax.experimental.pallas import tpu as pltpu
from jax.experimental.pallas import tpu_sc as plsc

N = 4096
D = 256
K = 2048
H = N // 2

_BI = 512
_BP = 512
_BM = 512


def _rank_body(h_ref, w_ref, b_ref, r_ref, s2_ref, sc_ref, sr_ref):
    i = pl.program_id(0)

    @pl.when(i == 0)
    def _():
        b0 = b_ref[0, 0]
        sc_ref[...] = jax.nn.sigmoid(
            jnp.dot(h_ref[...], w_ref[...],
                    preferred_element_type=jnp.float32) + b0)
        sr_ref[...] = jax.nn.sigmoid(
            lax.dot_general(w_ref[...], h_ref[...], (((0,), (1,)), ((), ())),
                            preferred_element_type=jnp.float32) + b0)
        s2_ref[...] = sr_ref[...]

    s_i = sc_ref[pl.ds(i * _BI, _BI), :]
    s_j = sr_ref[...]
    jj = lax.broadcasted_iota(jnp.int32, (_BI, N), 1)
    ii = i * _BI + lax.broadcasted_iota(jnp.int32, (_BI, N), 0)
    ahead = (s_j > s_i) | ((s_j == s_i) & (jj < ii))
    r_ref[...] = jnp.sum(ahead.astype(jnp.int32), axis=1, keepdims=True)


def _inv_body(r_row_ref, s_row_ref, idx_ref, val_ref):
    p = pl.program_id(0)
    rr = r_row_ref[...]
    ss = s_row_ref[...]
    pp = p * _BI + lax.broadcasted_iota(jnp.int32, (_BI, N), 0)
    jj = lax.broadcasted_iota(jnp.int32, (_BI, N), 1)
    m = rr == pp
    idx_ref[...] = jnp.sum(jnp.where(m, jj, 0), axis=1, keepdims=True)
    val_ref[...] = jnp.sum(jnp.where(m, ss, 0.0), axis=1, keepdims=True)


_RB = 256


def _prep_body(g_ref, up_ref, utp_ref):
    i = pl.program_id(0)
    graw = g_ref[...]
    lo = (graw[:, :H] != 0).astype(jnp.int32)
    hi = (graw[:, H:] != 0).astype(jnp.int32)
    up_ref[...] = lo + 2 * hi
    u_bf = (graw != 0).astype(jnp.bfloat16)
    eye = (lax.broadcasted_iota(jnp.int32, (_RB, _RB), 0)
           == lax.broadcasted_iota(jnp.int32, (_RB, _RB), 1)
           ).astype(jnp.bfloat16)
    ut = lax.dot_general(u_bf, eye, (((0,), (0,)), ((), ())),
                         preferred_element_type=jnp.float32)
    uti = ut.astype(jnp.int32)

    @pl.when(i % 2 == 0)
    def _():
        utp_ref[...] = uti

    @pl.when(i % 2 == 1)
    def _():
        utp_ref[...] += 2 * uti


def _mmf_body(r_ref, ct_ref, hg_ref, val_ref, g_ref, h_ref, deg_ref):
    j = pl.program_id(1)
    rp = r_ref[...]
    cp = ct_ref[...]
    rlo = (rp & 1).astype(jnp.bfloat16)
    rhi = (rp >> 1).astype(jnp.bfloat16)
    clo = (cp & 1).astype(jnp.bfloat16)
    chi = (cp >> 1).astype(jnp.bfloat16)
    dims = (((1,), (1,)), ((), ()))
    acc = lax.dot_general(rlo, clo, dims,
                          preferred_element_type=jnp.float32)
    acc = acc + lax.dot_general(rhi, chi, dims,
                                preferred_element_type=jnp.float32)
    bin_f = (acc != 0).astype(jnp.float32)
    part = jnp.sum(bin_f, axis=1, keepdims=True)

    @pl.when(j == 0)
    def _():
        deg_ref[...] = part
        g_ref[:, pl.ds(0, _BM)] = bin_f

    @pl.when((j != 0) & (j != 3))
    def _():
        deg_ref[...] += part
        g_ref[:, pl.ds(j * _BM, _BM)] = bin_f

    @pl.when(j == 3)
    def _():
        d = deg_ref[...] + part
        rec = 1.0 / jnp.where(d == 0, 1.0, d)
        g_ref[:, pl.ds(0, 3 * _BM)] *= rec
        g_ref[:, pl.ds(3 * _BM, _BM)] = bin_f * rec
        h_ref[...] = hg_ref[...] * val_ref[...]


def _make_sc_gather(nc, ns):
    nw = nc * ns
    rpw = K // nw
    chunk = rpw // 4
    mesh = plsc.VectorSubcoreMesh(core_axis_name="c", subcore_axis_name="s")

    @functools.partial(
        pl.kernel, mesh=mesh,
        out_type=[
            jax.ShapeDtypeStruct((K, H), jnp.int32),
            jax.ShapeDtypeStruct((K, H), jnp.int32),
            jax.ShapeDtypeStruct((K, D), jnp.float32),
        ],
        scratch_types=[
            pltpu.VMEM((rpw,), jnp.int32),
            pltpu.VMEM((chunk, H), jnp.int32),
            pltpu.VMEM((chunk, H), jnp.int32),
            pltpu.VMEM((rpw, D), jnp.float32),
            pltpu.SemaphoreType.DMA,
            pltpu.SemaphoreType.DMA,
            pltpu.SemaphoreType.DMA,
        ],
    )
    def sc_gather(up, utp, h, idx, rp_out, ctp_out, hg_out,
                  idx_v, buf0, buf1, hbuf, sem_g, sem_w0, sem_w1):
        wid = lax.axis_index("s") * nc + lax.axis_index("c")
        base = wid * rpw
        pltpu.sync_copy(idx.at[pl.ds(base, rpw)], idx_v)
        pltpu.async_copy(h.at[idx_v], hbuf, sem_g).wait()
        pltpu.sync_copy(hbuf, hg_out.at[pl.ds(base, rpw)])
        bufs = (buf0, buf1)
        wsems = (sem_w0, sem_w1)
        pending = [None, None]
        t = 0
        for src, dst in ((up, rp_out), (utp, ctp_out)):
            for c in range(4):
                sl = t % 2
                if pending[sl] is not None:
                    pending[sl].wait()
                iv = idx_v.at[pl.ds(c * chunk, chunk)]
                pltpu.async_copy(src.at[iv], bufs[sl], sem_g).wait()
                pending[sl] = pltpu.async_copy(
                    bufs[sl], dst.at[pl.ds(base + c * chunk, chunk)], wsems[sl])
                t += 1
        pending[0].wait()
        pending[1].wait()

    return sc_gather


def kernel(g, h, W, b):
    f32 = jnp.float32
    r, s2 = pl.pallas_call(
        _rank_body,
        grid=(N // _BI,),
        in_specs=[
            pl.BlockSpec((N, D), lambda i: (0, 0)),
            pl.BlockSpec((D, 1), lambda i: (0, 0)),
            pl.BlockSpec((1, 1), lambda i: (0, 0)),
        ],
        out_specs=[
            pl.BlockSpec((_BI, 1), lambda i: (i, 0)),
            pl.BlockSpec((1, N), lambda i: (0, 0)),
        ],
        out_shape=[
            jax.ShapeDtypeStruct((N, 1), jnp.int32),
            jax.ShapeDtypeStruct((1, N), f32),
        ],
        scratch_shapes=[
            pltpu.VMEM((N, 1), f32),
            pltpu.VMEM((1, N), f32),
        ],
    )(h, W, b.reshape(1, 1))
    r2 = r.reshape(1, N)

    idxf, vals = pl.pallas_call(
        _inv_body,
        grid=(K // _BI,),
        in_specs=[
            pl.BlockSpec((1, N), lambda i: (0, 0)),
            pl.BlockSpec((1, N), lambda i: (0, 0)),
        ],
        out_specs=[
            pl.BlockSpec((_BI, 1), lambda i: (i, 0)),
            pl.BlockSpec((_BI, 1), lambda i: (i, 0)),
        ],
        out_shape=[
            jax.ShapeDtypeStruct((K, 1), jnp.int32),
            jax.ShapeDtypeStruct((K, 1), f32),
        ],
    )(r2, s2)
    idx = idxf.reshape(K)

    up, utp = pl.pallas_call(
        _prep_body,
        grid=(N // _RB,),
        in_specs=[
            pl.BlockSpec((_RB, N), lambda i: ((i // 2) + (i % 2) * 8, 0)),
        ],
        out_specs=[
            pl.BlockSpec((_RB, H), lambda i: ((i // 2) + (i % 2) * 8, 0)),
            pl.BlockSpec((N, _RB), lambda i: (0, i // 2)),
        ],
        out_shape=[
            jax.ShapeDtypeStruct((N, H), jnp.int32),
            jax.ShapeDtypeStruct((N, H), jnp.int32),
        ],
    )(g)

    info = plsc.get_sparse_core_info()
    sc_gather = _make_sc_gather(info.num_cores, info.num_subcores)
    Rp, CTp, Hg = sc_gather(up, utp, h, idx)

    g_new, new_h = pl.pallas_call(
        _mmf_body,
        grid=(K // _BM, K // _BM),
        in_specs=[
            pl.BlockSpec((_BM, H), lambda i, j: (i, 0)),
            pl.BlockSpec((_BM, H), lambda i, j: (j, 0)),
            pl.BlockSpec((_BM, D), lambda i, j: (i, 0)),
            pl.BlockSpec((_BM, 1), lambda i, j: (i, 0)),
        ],
        out_specs=[
            pl.BlockSpec((_BM, K), lambda i, j: (i, 0)),
            pl.BlockSpec((_BM, D), lambda i, j: (i, 0)),
        ],
        out_shape=[
            jax.ShapeDtypeStruct((K, K), f32),
            jax.ShapeDtypeStruct((K, D), f32),
        ],
        scratch_shapes=[
            pltpu.VMEM((_BM, 1), f32),
        ],
    )(Rp, CTp, Hg, vals)

    return g_new, new_h, idx
